# baseline (speedup 1.0000x reference)
import functools

import jax
import jax.numpy as jnp
from jax import lax
from jax.experimental import pallas as pl
from jax.experimental.pallas import tpu as pltpu
from jax.experimental.pallas import tpu_sc as plsc

_NS = 16
_NC = 2
_LANES = 16
_GEDGE = 256
_IDXROWS = 4
_RCH = 128


def _ceil_to(x, m):
    return (x + m - 1) // m * m


def _edge_body(npad, ept_rows, dh, nrows_total, n,
               feat2, src2d, dst2d, acc_out, deg_out,
               acc, deg, sidx_a, didx_a, sidx_b, didx_b,
               rows_p, rows_q, ones_b,
               gsem_p, gsem_q, ssem_p, ssem_q, isem_a, isem_b):
    c = lax.axis_index("c")
    s = lax.axis_index("s")
    coff = c * n

    one_v = jnp.full((_LANES,), 1.0, jnp.float32)
    zero_v = jnp.zeros((_LANES,), jnp.float32)

    @pl.loop(0, _GEDGE)
    def _(i):
        ones_b[i, :] = zero_v

    @pl.loop(0, _RCH)
    def _(i):
        for q in range(dh // _LANES):
            rows_p[i, pl.ds(_LANES * q, _LANES)] = zero_v

    rpt = npad // _NS

    @pl.loop(0, rpt // _RCH)
    def _(k):
        r0 = s * rpt + k * _RCH
        pltpu.sync_copy(rows_p.at[pl.ds(0, _RCH)], acc.at[pl.ds(r0, _RCH)])
        pltpu.sync_copy(ones_b.at[pl.ds(0, _RCH)], deg.at[pl.ds(r0, _RCH)])

    @pl.loop(0, _GEDGE)
    def _(i):
        ones_b[i, :] = one_v

    plsc.subcore_barrier()

    ebase = s * ept_rows
    max_rb = nrows_total - _IDXROWS
    bufs = ((rows_p, gsem_p, ssem_p), (rows_q, gsem_q, ssem_q))

    def prime(sidx, didx, isem, rb):
        pltpu.async_copy(src2d.at[pl.ds(rb, _IDXROWS)], sidx, isem)
        pltpu.async_copy(dst2d.at[pl.ds(rb, _IDXROWS)], didx, isem)

    def wait_idx(sidx, didx, isem):
        pltpu.make_async_copy(src2d.at[pl.ds(0, _IDXROWS)], sidx, isem).wait()
        pltpu.make_async_copy(dst2d.at[pl.ds(0, _IDXROWS)], didx, isem).wait()

    def drain(buf, ssem):
        pltpu.make_async_copy(buf, acc.at[didx_a.at[0]], ssem).wait()
        pltpu.make_async_copy(ones_b, deg.at[didx_a.at[0]], ssem).wait()

    def drain_nodeg(buf, ssem):
        pltpu.make_async_copy(buf, acc.at[didx_a.at[0]], ssem).wait()

    def do_round(sidx, didx, j0, prev_descs, with_deg):
        gathers, descs = [], []
        for t, (buf, gsem, ssem) in enumerate(bufs):
            j = j0 + t
            if prev_descs is not None:
                for d_ in prev_descs[t]:
                    d_.wait()
            gathers.append(pltpu.async_copy(feat2.at[sidx.at[j]], buf, gsem))
            if with_deg:
                dg = pltpu.async_copy(ones_b, deg.at[didx.at[j]], ssem,
                                      add=True)
                descs.append([dg])
            else:
                descs.append([])
        for t, (buf, gsem, ssem) in enumerate(bufs):
            gathers[t].wait()
            descs[t].append(pltpu.async_copy(
                buf, acc.at[didx.at[j0 + t]], ssem, add=True))
        return descs

    def do_block(sidx, didx, isem, nsidx, ndidx, nisem, rb_next,
                 with_deg, prev_deg):
        wait_idx(sidx, didx, isem)
        dr = drain if prev_deg else drain_nodeg
        if prev_deg is not None:
            dr(rows_p, ssem_p)
            dr(rows_q, ssem_q)
        prime(nsidx, ndidx, nisem, rb_next)
        for j in range(_IDXROWS):
            for q in range(_GEDGE // _LANES):
                sl = pl.ds(_LANES * q, _LANES)
                sidx[j, sl] = sidx[j, sl] + coff
        descs = do_round(sidx, didx, 0, None, with_deg)
        do_round(sidx, didx, 2, descs, with_deg)

    half_rows = ept_rows // 2
    nblk = ept_rows // _IDXROWS

    def rbase(kb):
        return ebase + lax.rem(c * half_rows + kb * _IDXROWS,
                               jnp.int32(ept_rows))

    prime(sidx_a, didx_a, isem_a, rbase(0))
    do_block(sidx_a, didx_a, isem_a, sidx_b, didx_b, isem_b,
             rbase(1), True, None)
    do_block(sidx_b, didx_b, isem_b, sidx_a, didx_a, isem_a,
             rbase(2), True, True)

    @pl.loop(1, nblk // 4)
    def _(kb):
        do_block(sidx_a, didx_a, isem_a, sidx_b, didx_b, isem_b,
                 rbase(2 * kb + 1), True, True)
        do_block(sidx_b, didx_b, isem_b, sidx_a, didx_a, isem_a,
                 rbase(2 * kb + 2), True, True)

    do_block(sidx_a, didx_a, isem_a, sidx_b, didx_b, isem_b,
             rbase(nblk // 2 + 1), False, True)
    do_block(sidx_b, didx_b, isem_b, sidx_a, didx_a, isem_a,
             rbase(nblk // 2 + 2), False, False)

    @pl.loop(nblk // 4 + 1, nblk // 2)
    def _(kb):
        do_block(sidx_a, didx_a, isem_a, sidx_b, didx_b, isem_b,
                 rbase(2 * kb + 1), False, False)
        do_block(sidx_b, didx_b, isem_b, sidx_a, didx_a, isem_a,
                 rbase(2 * kb + 2), False, False)

    drain_nodeg(rows_p, ssem_p)
    drain_nodeg(rows_q, ssem_q)
    wait_idx(sidx_a, didx_a, isem_a)

    plsc.subcore_barrier()

    @pl.loop(0, rpt // _RCH)
    def _(k):
        r0 = s * rpt + k * _RCH
        pltpu.sync_copy(acc.at[pl.ds(r0, _RCH)], rows_p.at[pl.ds(0, _RCH)])
        pltpu.sync_copy(rows_p.at[pl.ds(0, _RCH)],
                        acc_out.at[c, pl.ds(r0, _RCH)])
        pltpu.sync_copy(deg.at[pl.ds(r0, _RCH)], ones_b.at[pl.ds(0, _RCH)])
        pltpu.sync_copy(ones_b.at[pl.ds(0, _RCH)],
                        deg_out.at[c, pl.ds(r0, _RCH)])


def _combine_body(feat_ref, a0_ref, a1_ref, d0_ref, d1_ref, out_ref):
    dtot = d0_ref[0][:, 0:1] + d1_ref[0][:, 0:1]
    inv = 1.0 / jnp.maximum(dtot, 1.0)
    agg = jnp.concatenate([a0_ref[0] * inv, a1_ref[0] * inv], axis=1)
    out_ref[...] = feat_ref[...] + agg


@jax.jit
def kernel(features, edge_index):
    n, d = features.shape
    e = edge_index.shape[1]
    dh = d // 2

    npad = _ceil_to(n + 1, _NS * _RCH)
    epad = _ceil_to(e, _NS * 2 * _IDXROWS * _GEDGE)

    ei = jnp.pad(edge_index.astype(jnp.int32), ((0, 0), (0, epad - e)),
                 constant_values=n)
    src2d = ei[1].reshape(-1, _GEDGE)
    dst2d = ei[0].reshape(-1, _GEDGE)

    feat2 = jnp.concatenate(
        [features[:, :dh], features[:, dh:],
         jnp.zeros((_LANES, dh), jnp.float32)], axis=0)

    ept_rows = src2d.shape[0] // _NS

    mesh = plsc.VectorSubcoreMesh(core_axis_name="c", subcore_axis_name="s")
    body = functools.partial(_edge_body, npad, ept_rows, dh,
                             src2d.shape[0], n)
    edge_kernel = pl.kernel(
        body,
        out_type=[jax.ShapeDtypeStruct((_NC, npad, dh), jnp.float32),
                  jax.ShapeDtypeStruct((_NC, npad, _LANES), jnp.float32)],
        mesh=mesh,
        compiler_params=pltpu.CompilerParams(use_tc_tiling_on_sc=False),
        scratch_types=[
            pltpu.VMEM_SHARED((npad, dh), jnp.float32),
            pltpu.VMEM_SHARED((npad, _LANES), jnp.float32),
            pltpu.VMEM((_IDXROWS, _GEDGE), jnp.int32),
            pltpu.VMEM((_IDXROWS, _GEDGE), jnp.int32),
            pltpu.VMEM((_IDXROWS, _GEDGE), jnp.int32),
            pltpu.VMEM((_IDXROWS, _GEDGE), jnp.int32),
            pltpu.VMEM((_GEDGE, dh), jnp.float32),
            pltpu.VMEM((_GEDGE, dh), jnp.float32),
            pltpu.VMEM((_GEDGE, _LANES), jnp.float32),
            pltpu.SemaphoreType.DMA,
            pltpu.SemaphoreType.DMA,
            pltpu.SemaphoreType.DMA,
            pltpu.SemaphoreType.DMA,
            pltpu.SemaphoreType.DMA,
            pltpu.SemaphoreType.DMA,
        ],
    )
    acc2, deg2 = edge_kernel(feat2, src2d, dst2d)

    blk = 2000
    out = pl.pallas_call(
        _combine_body,
        grid=(n // blk,),
        in_specs=[
            pl.BlockSpec((blk, d), lambda i: (i, 0)),
            pl.BlockSpec((1, blk, dh), lambda i: (0, i, 0)),
            pl.BlockSpec((1, blk, dh), lambda i: (1, i, 0)),
            pl.BlockSpec((1, blk, _LANES), lambda i: (0, i, 0)),
            pl.BlockSpec((1, blk, _LANES), lambda i: (1, i, 0)),
        ],
        out_specs=pl.BlockSpec((blk, d), lambda i: (i, 0)),
        out_shape=jax.ShapeDtypeStruct((n, d), jnp.float32),
    )(features, acc2, acc2, deg2, deg2)
    return out


# speedup vs baseline: 1.0983x; 1.0983x over previous
import functools

import jax
import jax.numpy as jnp
from jax import lax
from jax.experimental import pallas as pl
from jax.experimental.pallas import tpu as pltpu
from jax.experimental.pallas import tpu_sc as plsc

_NS = 16
_NC = 2
_LANES = 16
_GEDGE = 256
_IDXROWS = 4
_RCH = 128


def _ceil_to(x, m):
    return (x + m - 1) // m * m


def _edge_body(npad, ept_rows, dh, nrows_total,
               feat2, src2d, dst2d, acc_out, deg_out,
               acc, deg, sidx_a, didx_a, sidx_b, didx_b,
               rows_p, rows_q, ones_b,
               gsem_p, gsem_q, ssem_p, ssem_q, isem_a, isem_b):
    c = lax.axis_index("c")
    s = lax.axis_index("s")
    coff = c * npad

    one_v = jnp.full((_LANES,), 1.0, jnp.float32)
    zero_v = jnp.zeros((_LANES,), jnp.float32)

    @pl.loop(0, _GEDGE)
    def _(i):
        ones_b[i, :] = zero_v

    @pl.loop(0, _RCH)
    def _(i):
        for q in range(dh // _LANES):
            rows_p[i, pl.ds(_LANES * q, _LANES)] = zero_v

    rpt = npad // _NS

    @pl.loop(0, rpt // _RCH)
    def _(k):
        r0 = s * rpt + k * _RCH
        pltpu.sync_copy(rows_p.at[pl.ds(0, _RCH)], acc.at[pl.ds(r0, _RCH)])
        pltpu.sync_copy(ones_b.at[pl.ds(0, _RCH)], deg.at[pl.ds(r0, _RCH)])

    @pl.loop(0, _GEDGE)
    def _(i):
        ones_b[i, :] = one_v

    plsc.subcore_barrier()

    ebase = s * ept_rows
    max_rb = nrows_total - _IDXROWS
    bufs = ((rows_p, gsem_p, ssem_p), (rows_q, gsem_q, ssem_q))

    def prime(sidx, didx, isem, rb):
        pltpu.async_copy(src2d.at[pl.ds(rb, _IDXROWS)], sidx, isem)
        pltpu.async_copy(dst2d.at[pl.ds(rb, _IDXROWS)], didx, isem)

    def wait_idx(sidx, didx, isem):
        pltpu.make_async_copy(src2d.at[pl.ds(0, _IDXROWS)], sidx, isem).wait()
        pltpu.make_async_copy(dst2d.at[pl.ds(0, _IDXROWS)], didx, isem).wait()

    def drain(buf, ssem):
        pltpu.make_async_copy(buf, acc.at[didx_a.at[0]], ssem).wait()
        pltpu.make_async_copy(ones_b, deg.at[didx_a.at[0]], ssem).wait()

    def drain_nodeg(buf, ssem):
        pltpu.make_async_copy(buf, acc.at[didx_a.at[0]], ssem).wait()

    def do_round(sidx, didx, j0, prev_descs, with_deg):
        gathers, descs = [], []
        for t, (buf, gsem, ssem) in enumerate(bufs):
            j = j0 + t
            if prev_descs is not None:
                for d_ in prev_descs[t]:
                    d_.wait()
            gathers.append(pltpu.async_copy(feat2.at[sidx.at[j]], buf, gsem))
            if with_deg:
                dg = pltpu.async_copy(ones_b, deg.at[didx.at[j]], ssem,
                                      add=True)
                descs.append([dg])
            else:
                descs.append([])
        for t, (buf, gsem, ssem) in enumerate(bufs):
            gathers[t].wait()
            descs[t].append(pltpu.async_copy(
                buf, acc.at[didx.at[j0 + t]], ssem, add=True))
        return descs

    def do_block(sidx, didx, isem, nsidx, ndidx, nisem, rb_next,
                 with_deg, prev_deg):
        wait_idx(sidx, didx, isem)
        dr = drain if prev_deg else drain_nodeg
        if prev_deg is not None:
            dr(rows_p, ssem_p)
            dr(rows_q, ssem_q)
        prime(nsidx, ndidx, nisem, rb_next)
        for j in range(_IDXROWS):
            for q in range(_GEDGE // _LANES):
                sl = pl.ds(_LANES * q, _LANES)
                sidx[j, sl] = sidx[j, sl] + coff
        descs = do_round(sidx, didx, 0, None, with_deg)
        do_round(sidx, didx, 2, descs, with_deg)

    half_rows = ept_rows // 2
    nblk = ept_rows // _IDXROWS

    def rbase(kb):
        return ebase + lax.rem(c * half_rows + kb * _IDXROWS,
                               jnp.int32(ept_rows))

    prime(sidx_a, didx_a, isem_a, rbase(0))
    do_block(sidx_a, didx_a, isem_a, sidx_b, didx_b, isem_b,
             rbase(1), True, None)
    do_block(sidx_b, didx_b, isem_b, sidx_a, didx_a, isem_a,
             rbase(2), True, True)

    @pl.loop(1, nblk // 4)
    def _(kb):
        do_block(sidx_a, didx_a, isem_a, sidx_b, didx_b, isem_b,
                 rbase(2 * kb + 1), True, True)
        do_block(sidx_b, didx_b, isem_b, sidx_a, didx_a, isem_a,
                 rbase(2 * kb + 2), True, True)

    do_block(sidx_a, didx_a, isem_a, sidx_b, didx_b, isem_b,
             rbase(nblk // 2 + 1), False, True)
    do_block(sidx_b, didx_b, isem_b, sidx_a, didx_a, isem_a,
             rbase(nblk // 2 + 2), False, False)

    @pl.loop(nblk // 4 + 1, nblk // 2)
    def _(kb):
        do_block(sidx_a, didx_a, isem_a, sidx_b, didx_b, isem_b,
                 rbase(2 * kb + 1), False, False)
        do_block(sidx_b, didx_b, isem_b, sidx_a, didx_a, isem_a,
                 rbase(2 * kb + 2), False, False)

    drain_nodeg(rows_p, ssem_p)
    drain_nodeg(rows_q, ssem_q)
    wait_idx(sidx_a, didx_a, isem_a)

    plsc.subcore_barrier()

    @pl.loop(0, rpt // _RCH)
    def _(k):
        r0 = s * rpt + k * _RCH
        pltpu.sync_copy(acc.at[pl.ds(r0, _RCH)], rows_p.at[pl.ds(0, _RCH)])
        pltpu.sync_copy(rows_p.at[pl.ds(0, _RCH)],
                        acc_out.at[c, pl.ds(r0, _RCH)])
        pltpu.sync_copy(deg.at[pl.ds(r0, _RCH)], ones_b.at[pl.ds(0, _RCH)])
        pltpu.sync_copy(ones_b.at[pl.ds(0, _RCH)],
                        deg_out.at[c, pl.ds(r0, _RCH)])


def _combine_body(feat_ref, a0_ref, a1_ref, d0_ref, d1_ref, out_ref):
    dtot = d0_ref[0][:, 0:1] + d1_ref[0][:, 0:1]
    inv = 1.0 / jnp.maximum(dtot, 1.0)
    agg = jnp.concatenate([a0_ref[0] * inv, a1_ref[0] * inv], axis=1)
    out_ref[...] = feat_ref[...] + agg


@jax.jit
def kernel(features, edge_index):
    n, d = features.shape
    e = edge_index.shape[1]
    dh = d // 2

    npad = _ceil_to(n + 1, _NS * _RCH)
    epad = _ceil_to(e, _NS * 2 * _IDXROWS * _GEDGE)

    ei = jnp.pad(edge_index.astype(jnp.int32), ((0, 0), (0, epad - e)),
                 constant_values=n)
    src2d = ei[1].reshape(-1, _GEDGE)
    dst2d = ei[0].reshape(-1, _GEDGE)

    zrows = jnp.zeros((npad - n, dh), jnp.float32)
    feat2 = jnp.concatenate(
        [features[:, :dh], zrows, features[:, dh:], zrows], axis=0)

    ept_rows = src2d.shape[0] // _NS

    mesh = plsc.VectorSubcoreMesh(core_axis_name="c", subcore_axis_name="s")
    body = functools.partial(_edge_body, npad, ept_rows, dh, src2d.shape[0])
    edge_kernel = pl.kernel(
        body,
        out_type=[jax.ShapeDtypeStruct((_NC, npad, dh), jnp.float32),
                  jax.ShapeDtypeStruct((_NC, npad, _LANES), jnp.float32)],
        mesh=mesh,
        compiler_params=pltpu.CompilerParams(use_tc_tiling_on_sc=False),
        scratch_types=[
            pltpu.VMEM_SHARED((npad, dh), jnp.float32),
            pltpu.VMEM_SHARED((npad, _LANES), jnp.float32),
            pltpu.VMEM((_IDXROWS, _GEDGE), jnp.int32),
            pltpu.VMEM((_IDXROWS, _GEDGE), jnp.int32),
            pltpu.VMEM((_IDXROWS, _GEDGE), jnp.int32),
            pltpu.VMEM((_IDXROWS, _GEDGE), jnp.int32),
            pltpu.VMEM((_GEDGE, dh), jnp.float32),
            pltpu.VMEM((_GEDGE, dh), jnp.float32),
            pltpu.VMEM((_GEDGE, _LANES), jnp.float32),
            pltpu.SemaphoreType.DMA,
            pltpu.SemaphoreType.DMA,
            pltpu.SemaphoreType.DMA,
            pltpu.SemaphoreType.DMA,
            pltpu.SemaphoreType.DMA,
            pltpu.SemaphoreType.DMA,
        ],
    )
    acc2, deg2 = edge_kernel(feat2, src2d, dst2d)

    blk = 2000
    out = pl.pallas_call(
        _combine_body,
        grid=(n // blk,),
        in_specs=[
            pl.BlockSpec((blk, d), lambda i: (i, 0)),
            pl.BlockSpec((1, blk, dh), lambda i: (0, i, 0)),
            pl.BlockSpec((1, blk, dh), lambda i: (1, i, 0)),
            pl.BlockSpec((1, blk, _LANES), lambda i: (0, i, 0)),
            pl.BlockSpec((1, blk, _LANES), lambda i: (1, i, 0)),
        ],
        out_specs=pl.BlockSpec((blk, d), lambda i: (i, 0)),
        out_shape=jax.ShapeDtypeStruct((n, d), jnp.float32),
    )(features, acc2, acc2, deg2, deg2)
    return out


# speedup vs baseline: 1.1055x; 1.0066x over previous
import functools

import jax
import jax.numpy as jnp
from jax import lax
from jax.experimental import pallas as pl
from jax.experimental.pallas import tpu as pltpu
from jax.experimental.pallas import tpu_sc as plsc

_NS = 16
_NC = 2
_LANES = 16
_GEDGE = 320
_IDXROWS = 4
_RCH = 128


def _ceil_to(x, m):
    return (x + m - 1) // m * m


def _edge_body(npad, ept_rows, dh, nrows_total,
               feat2, src2d, dst2d, acc_out, deg_out,
               acc, deg, sidx_a, didx_a, sidx_b, didx_b,
               rows_p, rows_q, ones_b,
               gsem_p, gsem_q, ssem_p, ssem_q, isem_a, isem_b):
    c = lax.axis_index("c")
    s = lax.axis_index("s")
    coff = c * npad

    one_v = jnp.full((_LANES,), 1.0, jnp.float32)
    zero_v = jnp.zeros((_LANES,), jnp.float32)

    @pl.loop(0, _GEDGE)
    def _(i):
        ones_b[i, :] = zero_v

    @pl.loop(0, _RCH)
    def _(i):
        for q in range(dh // _LANES):
            rows_p[i, pl.ds(_LANES * q, _LANES)] = zero_v

    rpt = npad // _NS

    @pl.loop(0, rpt // _RCH)
    def _(k):
        r0 = s * rpt + k * _RCH
        pltpu.sync_copy(rows_p.at[pl.ds(0, _RCH)], acc.at[pl.ds(r0, _RCH)])
        pltpu.sync_copy(ones_b.at[pl.ds(0, _RCH)], deg.at[pl.ds(r0, _RCH)])

    @pl.loop(0, _GEDGE)
    def _(i):
        ones_b[i, :] = one_v

    plsc.subcore_barrier()

    ebase = s * ept_rows
    max_rb = nrows_total - _IDXROWS
    bufs = ((rows_p, gsem_p, ssem_p), (rows_q, gsem_q, ssem_q))

    def prime(sidx, didx, isem, rb):
        pltpu.async_copy(src2d.at[pl.ds(rb, _IDXROWS)], sidx, isem)
        pltpu.async_copy(dst2d.at[pl.ds(rb, _IDXROWS)], didx, isem)

    def wait_idx(sidx, didx, isem):
        pltpu.make_async_copy(src2d.at[pl.ds(0, _IDXROWS)], sidx, isem).wait()
        pltpu.make_async_copy(dst2d.at[pl.ds(0, _IDXROWS)], didx, isem).wait()

    def drain(buf, ssem):
        pltpu.make_async_copy(buf, acc.at[didx_a.at[0]], ssem).wait()
        pltpu.make_async_copy(ones_b, deg.at[didx_a.at[0]], ssem).wait()

    def drain_nodeg(buf, ssem):
        pltpu.make_async_copy(buf, acc.at[didx_a.at[0]], ssem).wait()

    def do_round(sidx, didx, j0, prev_descs, with_deg):
        gathers, descs = [], []
        for t, (buf, gsem, ssem) in enumerate(bufs):
            j = j0 + t
            if prev_descs is not None:
                for d_ in prev_descs[t]:
                    d_.wait()
            gathers.append(pltpu.async_copy(feat2.at[sidx.at[j]], buf, gsem))
            if with_deg:
                dg = pltpu.async_copy(ones_b, deg.at[didx.at[j]], ssem,
                                      add=True)
                descs.append([dg])
            else:
                descs.append([])
        for t, (buf, gsem, ssem) in enumerate(bufs):
            gathers[t].wait()
            descs[t].append(pltpu.async_copy(
                buf, acc.at[didx.at[j0 + t]], ssem, add=True))
        return descs

    def do_block(sidx, didx, isem, nsidx, ndidx, nisem, rb_next,
                 with_deg, prev_deg):
        wait_idx(sidx, didx, isem)
        dr = drain if prev_deg else drain_nodeg
        if prev_deg is not None:
            dr(rows_p, ssem_p)
            dr(rows_q, ssem_q)
        prime(nsidx, ndidx, nisem, rb_next)
        for j in range(_IDXROWS):
            for q in range(_GEDGE // _LANES):
                sl = pl.ds(_LANES * q, _LANES)
                sidx[j, sl] = sidx[j, sl] + coff
        descs = do_round(sidx, didx, 0, None, with_deg)
        do_round(sidx, didx, 2, descs, with_deg)

    half_rows = ept_rows // 2
    nblk = ept_rows // _IDXROWS

    def rbase(kb):
        return ebase + lax.rem(c * half_rows + kb * _IDXROWS,
                               jnp.int32(ept_rows))

    prime(sidx_a, didx_a, isem_a, rbase(0))
    do_block(sidx_a, didx_a, isem_a, sidx_b, didx_b, isem_b,
             rbase(1), True, None)
    do_block(sidx_b, didx_b, isem_b, sidx_a, didx_a, isem_a,
             rbase(2), True, True)

    @pl.loop(1, nblk // 4)
    def _(kb):
        do_block(sidx_a, didx_a, isem_a, sidx_b, didx_b, isem_b,
                 rbase(2 * kb + 1), True, True)
        do_block(sidx_b, didx_b, isem_b, sidx_a, didx_a, isem_a,
                 rbase(2 * kb + 2), True, True)

    do_block(sidx_a, didx_a, isem_a, sidx_b, didx_b, isem_b,
             rbase(nblk // 2 + 1), False, True)
    do_block(sidx_b, didx_b, isem_b, sidx_a, didx_a, isem_a,
             rbase(nblk // 2 + 2), False, False)

    @pl.loop(nblk // 4 + 1, nblk // 2)
    def _(kb):
        do_block(sidx_a, didx_a, isem_a, sidx_b, didx_b, isem_b,
                 rbase(2 * kb + 1), False, False)
        do_block(sidx_b, didx_b, isem_b, sidx_a, didx_a, isem_a,
                 rbase(2 * kb + 2), False, False)

    drain_nodeg(rows_p, ssem_p)
    drain_nodeg(rows_q, ssem_q)
    wait_idx(sidx_a, didx_a, isem_a)

    plsc.subcore_barrier()

    @pl.loop(0, rpt // _RCH)
    def _(k):
        r0 = s * rpt + k * _RCH
        pltpu.sync_copy(acc.at[pl.ds(r0, _RCH)], rows_p.at[pl.ds(0, _RCH)])
        pltpu.sync_copy(rows_p.at[pl.ds(0, _RCH)],
                        acc_out.at[c, pl.ds(r0, _RCH)])
        pltpu.sync_copy(deg.at[pl.ds(r0, _RCH)], ones_b.at[pl.ds(0, _RCH)])
        pltpu.sync_copy(ones_b.at[pl.ds(0, _RCH)],
                        deg_out.at[c, pl.ds(r0, _RCH)])


def _combine_body(feat_ref, a0_ref, a1_ref, d0_ref, d1_ref, out_ref):
    dtot = d0_ref[0][:, 0:1] + d1_ref[0][:, 0:1]
    inv = 1.0 / jnp.maximum(dtot, 1.0)
    agg = jnp.concatenate([a0_ref[0] * inv, a1_ref[0] * inv], axis=1)
    out_ref[...] = feat_ref[...] + agg


@jax.jit
def kernel(features, edge_index):
    n, d = features.shape
    e = edge_index.shape[1]
    dh = d // 2

    npad = _ceil_to(n + 1, _NS * _RCH)
    epad = _ceil_to(e, _NS * 2 * _IDXROWS * _GEDGE)

    ei = jnp.pad(edge_index.astype(jnp.int32), ((0, 0), (0, epad - e)),
                 constant_values=n)
    src2d = ei[1].reshape(-1, _GEDGE)
    dst2d = ei[0].reshape(-1, _GEDGE)

    zrows = jnp.zeros((npad - n, dh), jnp.float32)
    feat2 = jnp.concatenate(
        [features[:, :dh], zrows, features[:, dh:], zrows], axis=0)

    ept_rows = src2d.shape[0] // _NS

    mesh = plsc.VectorSubcoreMesh(core_axis_name="c", subcore_axis_name="s")
    body = functools.partial(_edge_body, npad, ept_rows, dh, src2d.shape[0])
    edge_kernel = pl.kernel(
        body,
        out_type=[jax.ShapeDtypeStruct((_NC, npad, dh), jnp.float32),
                  jax.ShapeDtypeStruct((_NC, npad, _LANES), jnp.float32)],
        mesh=mesh,
        compiler_params=pltpu.CompilerParams(use_tc_tiling_on_sc=False),
        scratch_types=[
            pltpu.VMEM_SHARED((npad, dh), jnp.float32),
            pltpu.VMEM_SHARED((npad, _LANES), jnp.float32),
            pltpu.VMEM((_IDXROWS, _GEDGE), jnp.int32),
            pltpu.VMEM((_IDXROWS, _GEDGE), jnp.int32),
            pltpu.VMEM((_IDXROWS, _GEDGE), jnp.int32),
            pltpu.VMEM((_IDXROWS, _GEDGE), jnp.int32),
            pltpu.VMEM((_GEDGE, dh), jnp.float32),
            pltpu.VMEM((_GEDGE, dh), jnp.float32),
            pltpu.VMEM((_GEDGE, _LANES), jnp.float32),
            pltpu.SemaphoreType.DMA,
            pltpu.SemaphoreType.DMA,
            pltpu.SemaphoreType.DMA,
            pltpu.SemaphoreType.DMA,
            pltpu.SemaphoreType.DMA,
            pltpu.SemaphoreType.DMA,
        ],
    )
    acc2, deg2 = edge_kernel(feat2, src2d, dst2d)

    blk = 2000
    out = pl.pallas_call(
        _combine_body,
        grid=(n // blk,),
        in_specs=[
            pl.BlockSpec((blk, d), lambda i: (i, 0)),
            pl.BlockSpec((1, blk, dh), lambda i: (0, i, 0)),
            pl.BlockSpec((1, blk, dh), lambda i: (1, i, 0)),
            pl.BlockSpec((1, blk, _LANES), lambda i: (0, i, 0)),
            pl.BlockSpec((1, blk, _LANES), lambda i: (1, i, 0)),
        ],
        out_specs=pl.BlockSpec((blk, d), lambda i: (i, 0)),
        out_shape=jax.ShapeDtypeStruct((n, d), jnp.float32),
    )(features, acc2, acc2, deg2, deg2)
    return out
